# baseline (speedup 1.0000x reference)
import functools

import jax
import jax.numpy as jnp
from jax.experimental import pallas as pl
from jax.experimental.pallas import tpu as pltpu
from jax.experimental.pallas import tpu_sc as plsc

_CAPACITY = 0.5


def _select(s, k):
    Bn, S, L = s.shape
    weights = jax.nn.sigmoid(s)

    b32 = jax.lax.bitcast_convert_type(s, jnp.int32)
    mag = b32 & jnp.int32(0x7FFFFFFF)
    keys = jnp.where(b32 >= 0, b32, jnp.int32(-1) - mag)

    def count_ge(c):
        return jnp.sum((keys >= c).astype(jnp.int32), axis=(1, 2),
                       keepdims=True)

    thr = jnp.where(count_ge(jnp.int32(0)) >= k,
                    jnp.int32(0), jnp.int32(-(2**31)))
    for hi_bit in range(30, 0, -2):
        q = jnp.int32(1 << (hi_bit - 1))
        d1 = (count_ge(thr + q) >= k).astype(jnp.int32)
        d2 = (count_ge(thr + 2 * q) >= k).astype(jnp.int32)
        d3 = (count_ge(thr + 3 * q) >= k).astype(jnp.int32)
        thr = thr + q * (d1 + d2 + d3)
    thr = jnp.where(count_ge(thr + 1) >= k, thr + 1, thr)

    gt = keys > thr
    eq = keys == thr
    cnt_gt = jnp.sum(gt.astype(jnp.int32), axis=(1, 2), keepdims=True)
    cnt_eq = jnp.sum(eq.astype(jnp.int32), axis=(1, 2), keepdims=True)
    need = k - cnt_gt

    pos = (jax.lax.broadcasted_iota(jnp.int32, (Bn, S, L), 1) * L
           + jax.lax.broadcasted_iota(jnp.int32, (Bn, S, L), 2))
    T = S * L

    def tie_search():
        lo = jnp.zeros((Bn, 1, 1), jnp.int32)
        hi = jnp.full((Bn, 1, 1), T, jnp.int32)
        for _ in range((T.bit_length() + 1) // 2 + 1):
            w = hi - lo
            m1, m2, m3 = lo + w // 4, lo + w // 2, lo + (3 * w) // 4
            c1 = jnp.sum((eq & (pos < m1)).astype(jnp.int32), axis=(1, 2),
                         keepdims=True) >= need
            c2 = jnp.sum((eq & (pos < m2)).astype(jnp.int32), axis=(1, 2),
                         keepdims=True) >= need
            c3 = jnp.sum((eq & (pos < m3)).astype(jnp.int32), axis=(1, 2),
                         keepdims=True) >= need
            hi = jnp.where(c1, m1, jnp.where(c2, m2, jnp.where(c3, m3, hi)))
            lo = jnp.where(~c3, m3, jnp.where(~c2, m2, jnp.where(~c1, m1, lo)))
        return hi

    no_ties = jnp.all(need == cnt_eq)
    hi = jax.lax.cond(no_ties,
                      lambda: jnp.full((Bn, 1, 1), T, jnp.int32),
                      tie_search)
    return gt | (eq & (pos < hi)), weights


_NS = 2
_TT = 512


def _fused_kernel(*refs, k, nsteps, sub, ns):
    x_refs = refs[:ns]
    w_ref, b_ref, mask_ref, wout_ref, sc_ref = refs[ns:]
    i = pl.program_id(0)
    for j in range(ns):
        s = jax.lax.dot_general(
            w_ref[...], x_refs[j][...],
            dimension_numbers=(((1,), (1,)), ((), ())),
            preferred_element_type=jnp.float32,
        ) + b_ref[0, 0]
        a = i * ns + j
        sc_ref[a // sub, a % sub, :] = s[0]

    @pl.when(i == nsteps - 1)
    def _():
        mask, weights = _select(sc_ref[...], k)
        mask_ref[...] = mask
        wout_ref[...] = weights


_NW = 32


def _sc_stream_probe(xr, sc_rows, rb):
    nrows, D = xr.shape
    per_tile = sc_rows // _NW
    mesh = plsc.VectorSubcoreMesh(core_axis_name="c", subcore_axis_name="s")

    @functools.partial(
        pl.kernel,
        out_type=jax.ShapeDtypeStruct((_NW, 16), jnp.float32),
        mesh=mesh,
        scratch_types=[pltpu.VMEM((rb, D), jnp.float32)],
    )
    def body(x_hbm, out_hbm, buf):
        c = jax.lax.axis_index("c")
        s = jax.lax.axis_index("s")
        wid = s * 2 + c
        base = (nrows - sc_rows) + wid * per_tile

        def step(i, carry):
            pltpu.sync_copy(x_hbm.at[pl.ds(base + i * rb, rb)], buf)
            return carry
        jax.lax.fori_loop(0, per_tile // rb, step, 0)
        pltpu.sync_copy(buf.at[0, pl.ds(0, 16)], out_hbm.at[wid])

    return body(xr)


def kernel(x, W, b):
    B, T, D = x.shape
    k = max(1, int(T * _CAPACITY))

    Tt, ns = _TT, _NS
    nsteps = (B * T) // (Tt * ns)
    sub = T // Tt
    xr = x.reshape(B * T, D)
    b2 = b.reshape(1, 1)

    def mk_spec(j):
        return pl.BlockSpec((Tt, D), lambda i: (i * ns + j, 0))

    mask3, w3 = pl.pallas_call(
        functools.partial(_fused_kernel, k=k, nsteps=nsteps, sub=sub, ns=ns),
        grid=(nsteps,),
        in_specs=[mk_spec(j) for j in range(ns)] + [
            pl.BlockSpec((1, D), lambda i: (0, 0)),
            pl.BlockSpec((1, 1), lambda i: (0, 0)),
        ],
        out_specs=(
            pl.BlockSpec((B, sub, Tt), lambda i: (0, 0, 0)),
            pl.BlockSpec((B, sub, Tt), lambda i: (0, 0, 0)),
        ),
        out_shape=(
            jax.ShapeDtypeStruct((B, sub, Tt), jnp.bool_),
            jax.ShapeDtypeStruct((B, sub, Tt), jnp.float32),
        ),
        scratch_shapes=[pltpu.VMEM((B, sub, Tt), jnp.float32)],
    )(*([xr] * ns), W, b2)
    junk = _sc_stream_probe(xr, 14336, 16)

    def _combine(w_ref, j_ref, o_ref):
        o_ref[...] = w_ref[...] + 0.0 * j_ref[0, 0]

    w_out = pl.pallas_call(
        _combine,
        out_shape=jax.ShapeDtypeStruct((B, sub, Tt), jnp.float32),
    )(w3, junk)
    return (mask3.reshape(B, T), w_out.reshape(B, T))


# speedup vs baseline: 1.5676x; 1.5676x over previous
import functools

import jax
import jax.numpy as jnp
from jax.experimental import pallas as pl
from jax.experimental.pallas import tpu as pltpu

_CAPACITY = 0.5


def _select(s, k):
    Bn, S, L = s.shape
    weights = jax.nn.sigmoid(s)

    b32 = jax.lax.bitcast_convert_type(s, jnp.int32)
    mag = b32 & jnp.int32(0x7FFFFFFF)
    keys = jnp.where(b32 >= 0, b32, jnp.int32(-1) - mag)

    def count_ge(c):
        return jnp.sum((keys >= c).astype(jnp.int32), axis=(1, 2),
                       keepdims=True)

    thr = jnp.where(count_ge(jnp.int32(0)) >= k,
                    jnp.int32(0), jnp.int32(-(2**31)))
    for hi_bit in range(30, 0, -2):
        q = jnp.int32(1 << (hi_bit - 1))
        d1 = (count_ge(thr + q) >= k).astype(jnp.int32)
        d2 = (count_ge(thr + 2 * q) >= k).astype(jnp.int32)
        d3 = (count_ge(thr + 3 * q) >= k).astype(jnp.int32)
        thr = thr + q * (d1 + d2 + d3)
    thr = jnp.where(count_ge(thr + 1) >= k, thr + 1, thr)

    gt = keys > thr
    eq = keys == thr
    cnt_gt = jnp.sum(gt.astype(jnp.int32), axis=(1, 2), keepdims=True)
    cnt_eq = jnp.sum(eq.astype(jnp.int32), axis=(1, 2), keepdims=True)
    need = k - cnt_gt

    pos = (jax.lax.broadcasted_iota(jnp.int32, (Bn, S, L), 1) * L
           + jax.lax.broadcasted_iota(jnp.int32, (Bn, S, L), 2))
    T = S * L

    def tie_search():
        lo = jnp.zeros((Bn, 1, 1), jnp.int32)
        hi = jnp.full((Bn, 1, 1), T, jnp.int32)
        for _ in range((T.bit_length() + 1) // 2 + 1):
            w = hi - lo
            m1, m2, m3 = lo + w // 4, lo + w // 2, lo + (3 * w) // 4
            c1 = jnp.sum((eq & (pos < m1)).astype(jnp.int32), axis=(1, 2),
                         keepdims=True) >= need
            c2 = jnp.sum((eq & (pos < m2)).astype(jnp.int32), axis=(1, 2),
                         keepdims=True) >= need
            c3 = jnp.sum((eq & (pos < m3)).astype(jnp.int32), axis=(1, 2),
                         keepdims=True) >= need
            hi = jnp.where(c1, m1, jnp.where(c2, m2, jnp.where(c3, m3, hi)))
            lo = jnp.where(~c3, m3, jnp.where(~c2, m2, jnp.where(~c1, m1, lo)))
        return hi

    no_ties = jnp.all(need == cnt_eq)
    hi = jax.lax.cond(no_ties,
                      lambda: jnp.full((Bn, 1, 1), T, jnp.int32),
                      tie_search)
    return gt | (eq & (pos < hi)), weights


_NS = 2
_TT = 512


def _fused_kernel(*refs, k, nsteps, sub, ns):
    x_refs = refs[:ns]
    w_ref, b_ref, mask_ref, wout_ref, sc_ref = refs[ns:]
    i = pl.program_id(0)
    for j in range(ns):
        s = jax.lax.dot_general(
            w_ref[...], x_refs[j][...],
            dimension_numbers=(((1,), (1,)), ((), ())),
            preferred_element_type=jnp.float32,
        ) + b_ref[0, 0]
        a = i * ns + j
        sc_ref[a // sub, a % sub, :] = s[0]

    @pl.when(i == nsteps - 1)
    def _():
        mask, weights = _select(sc_ref[...], k)
        mask_ref[...] = mask
        wout_ref[...] = weights


def kernel(x, W, b):
    B, T, D = x.shape
    k = max(1, int(T * _CAPACITY))

    Tt, ns = _TT, _NS
    nsteps = (B * T) // (Tt * ns)
    sub = T // Tt
    xr = x.reshape(B * T, D)
    b2 = b.reshape(1, 1)

    def mk_spec(j):
        return pl.BlockSpec((Tt, D), lambda i: (i * ns + j, 0))

    mask3, w3 = pl.pallas_call(
        functools.partial(_fused_kernel, k=k, nsteps=nsteps, sub=sub, ns=ns),
        grid=(nsteps,),
        in_specs=[mk_spec(j) for j in range(ns)] + [
            pl.BlockSpec((1, D), lambda i: (0, 0)),
            pl.BlockSpec((1, 1), lambda i: (0, 0)),
        ],
        out_specs=(
            pl.BlockSpec((B, sub, Tt), lambda i: (0, 0, 0)),
            pl.BlockSpec((B, sub, Tt), lambda i: (0, 0, 0)),
        ),
        out_shape=(
            jax.ShapeDtypeStruct((B, sub, Tt), jnp.bool_),
            jax.ShapeDtypeStruct((B, sub, Tt), jnp.float32),
        ),
        scratch_shapes=[pltpu.VMEM((B, sub, Tt), jnp.float32)],
    )(*([xr] * ns), W, b2)
    return (mask3.reshape(B, T), w3.reshape(B, T))
